# baseline (speedup 1.0000x reference)
import functools
import math

import jax
import jax.numpy as jnp
from jax import lax
from jax.experimental import pallas as pl
from jax.experimental.pallas import tpu as pltpu
from jax.experimental.pallas import tpu_sc as plsc

NC = 2
NS = 16
NW = NC * NS

CH = 128
NBUF = 7
H = 2
GSEQ = 32


def _sc_gather_flat(table, idx):
    NP, = idx.shape
    V, D = table.shape
    npw = NP // NW
    nchunk = npw // CH
    nouter = -(-nchunk // NBUF)
    nprime = min(NBUF, nchunk)
    mesh = plsc.VectorSubcoreMesh(core_axis_name="c", subcore_axis_name="s")

    @functools.partial(
        pl.kernel,
        out_type=jax.ShapeDtypeStruct((NP, D), jnp.float32),
        mesh=mesh,
        scratch_types=[
            pltpu.VMEM((npw,), jnp.int32),
            pltpu.VMEM((NBUF, CH, D), jnp.float32),
            pltpu.SemaphoreType.DMA((NBUF,)),
        ],
    )
    def run(table_hbm, idx_hbm, out_hbm, idx_v, rows_v, sems):
        wid = lax.axis_index("s") * NC + lax.axis_index("c")
        rbase = wid * npw
        pltpu.sync_copy(idx_hbm.at[pl.ds(rbase, npw)], idx_v)
        for b in range(nprime):
            pltpu.async_copy(
                table_hbm.at[idx_v.at[pl.ds(b * CH, CH)]], rows_v.at[b], sems.at[b]
            )

        def outer(g, carry):
            for b in range(NBUF):
                j = g * NBUF + b

                @pl.when(j < nchunk)
                def _():
                    pltpu.make_async_copy(
                        table_hbm.at[idx_v.at[pl.ds(0, CH)]], rows_v.at[b], sems.at[b]
                    ).wait()
                    pltpu.sync_copy(
                        rows_v.at[b], out_hbm.at[pl.ds(rbase + j * CH, CH)]
                    )
                    jn = j + NBUF

                    @pl.when(jn < nchunk)
                    def _():
                        pltpu.async_copy(
                            table_hbm.at[idx_v.at[pl.ds(jn * CH, CH)]],
                            rows_v.at[b], sems.at[b]
                        )
            return carry

        lax.fori_loop(0, nouter, outer, 0)

    return run(table, idx)


def _make_scale_reshape_body(scale, S, gseq, with_alias):
    def body(*refs):
        if with_alias:
            _, x_ref, o_ref = refs
        else:
            x_ref, o_ref = refs
        for g in range(gseq):
            o_ref[g] = x_ref[pl.ds(g * S, S), :] * scale
    return body


def _scale_reshape(flat, out_prev, scale, S, D, blk_off, total_b):
    rows = flat.shape[0]
    nb = rows // S
    nblk = nb // GSEQ
    body = _make_scale_reshape_body(scale, S, GSEQ, out_prev is not None)
    out_spec = pl.BlockSpec((GSEQ, S, D), lambda i: (i + blk_off, 0, 0))
    flat_spec = pl.BlockSpec((GSEQ * S, D), lambda i: (i, 0))
    if out_prev is None:
        return pl.pallas_call(
            body,
            grid=(nblk,),
            in_specs=[flat_spec],
            out_specs=out_spec,
            out_shape=jax.ShapeDtypeStruct((total_b, S, D), jnp.float32),
        )(flat)
    return pl.pallas_call(
        body,
        grid=(nblk,),
        in_specs=[pl.BlockSpec(memory_space=pl.ANY), flat_spec],
        out_specs=out_spec,
        out_shape=jax.ShapeDtypeStruct((total_b, S, D), jnp.float32),
        input_output_aliases={0: 0},
    )(out_prev, flat)


def kernel(tokens, embedding):
    B, S = tokens.shape
    V, D = embedding.shape
    N = B * S
    scale = math.sqrt(D)
    idx = tokens.reshape(N).astype(jnp.int32)

    span = NW * CH
    h = H if (N % H == 0 and (N // H) % span == 0 and (B // H) % GSEQ == 0
              and B % H == 0) else 1
    if h == 1 and (N % span != 0 or B % GSEQ != 0):
        NP = -(-N // span) * span
        if NP != N:
            idx = jnp.concatenate([idx, jnp.zeros((NP - N,), jnp.int32)])
        out = _sc_gather_flat(embedding, idx)
        return (out[:N] * scale).reshape(B, S, D)

    nh = N // h
    bh = B // h
    out = None
    for i in range(h):
        flat = _sc_gather_flat(embedding, lax.slice(idx, (i * nh,), ((i + 1) * nh,)))
        out = _scale_reshape(flat, out, scale, S, D, i * (bh // GSEQ), B)
    return out
